# baseline (speedup 1.0000x reference)
import functools

import jax
import jax.numpy as jnp
from jax import lax
from jax.experimental import pallas as pl
from jax.experimental.pallas import tpu as pltpu
from jax.experimental.pallas import tpu_sc as plsc

N_NODES = 10000
D = 128
DA = 136
E_EDGES = 320000

NC = 2
NS = 16
NW = NC * NS
CHUNK = 128
NBUF = 2
NCH = 80
EPT = NCH * CHUNK
E_PAD = EPT * NW
ACC_N = 10112
ZPT = ACC_N // NS

_vector_mesh = plsc.VectorSubcoreMesh(core_axis_name="c", subcore_axis_name="s")


@functools.partial(
    pl.kernel,
    out_type=jax.ShapeDtypeStruct((NC, ACC_N, DA), jnp.float32),
    mesh=_vector_mesh,
    scratch_types=[
        pltpu.VMEM((NBUF, CHUNK), jnp.int32),
        pltpu.VMEM((NBUF, CHUNK), jnp.int32),
        pltpu.VMEM((NBUF, CHUNK, DA), jnp.float32),
        pltpu.VMEM_SHARED((ACC_N, DA), jnp.float32),
        pltpu.SemaphoreType.DMA,
        pltpu.SemaphoreType.DMA,
        pltpu.SemaphoreType.DMA,
        pltpu.SemaphoreType.DMA,
    ],
    compiler_params=pltpu.CompilerParams(use_tc_tiling_on_sc=False),
)
def _sc_aggregate(xa_hbm, src_hbm, dst_hbm, zero_hbm, sum_hbm,
                  src_r, dst_r, rows_r, acc_sh, si0, si1, sg0, sg1):
    si = (si0, si1)
    sg = (sg0, sg1)
    cid = lax.axis_index("c")
    sid = lax.axis_index("s")
    wid = cid * NS + sid

    zbase = sid * ZPT
    pltpu.sync_copy(zero_hbm, acc_sh.at[pl.ds(zbase, ZPT)])

    for b in range(NBUF):
        pltpu.async_copy(src_hbm.at[wid].at[b], src_r.at[b], si[b])
        pltpu.async_copy(dst_hbm.at[wid].at[b], dst_r.at[b], si[b])
    pltpu.make_async_copy(src_hbm.at[wid].at[0], src_r.at[0], si[0]).wait()
    pltpu.async_copy(xa_hbm.at[src_r.at[0]], rows_r.at[0], sg[0])

    plsc.subcore_barrier()

    @pl.loop(0, NCH, step=NBUF)
    def _(j):
        for b in range(NBUF):
            c = j + b
            nb = 1 - b
            pltpu.make_async_copy(
                xa_hbm.at[src_r.at[b]], rows_r.at[b], sg[b]).wait()

            @pl.when(c + 1 < NCH)
            def _():
                pltpu.make_async_copy(
                    src_hbm.at[wid].at[c + 1], src_r.at[nb], si[nb]).wait()
                pltpu.async_copy(xa_hbm.at[src_r.at[nb]], rows_r.at[nb], sg[nb])

            pltpu.make_async_copy(
                dst_hbm.at[wid].at[c], dst_r.at[b], si[b]).wait()
            pltpu.sync_copy(rows_r.at[b], acc_sh.at[dst_r.at[b]], add=True)

            @pl.when(c + 2 < NCH)
            def _():
                pltpu.async_copy(src_hbm.at[wid].at[c + 2], src_r.at[b], si[b])
                pltpu.async_copy(dst_hbm.at[wid].at[c + 2], dst_r.at[b], si[b])

    plsc.subcore_barrier()

    pltpu.sync_copy(acc_sh.at[pl.ds(zbase, ZPT)],
                    sum_hbm.at[cid].at[pl.ds(zbase, ZPT)])


BLK = 400


def _combine_body(sum_ref, x_ref, wl_ref, wr_ref, bl_ref, o_ref):
    a = sum_ref[0] + sum_ref[1]
    s = a[:, :D]
    c = a[:, D:D + 1]
    mean = s / jnp.maximum(c, 1.0)
    o_ref[...] = (
        jnp.dot(mean, wl_ref[...], preferred_element_type=jnp.float32)
        + jnp.dot(x_ref[...], wr_ref[...], preferred_element_type=jnp.float32)
        + bl_ref[...]
    )


def _combine(sums, x, wl_t, wr_t, bl):
    return pl.pallas_call(
        _combine_body,
        grid=(N_NODES // BLK,),
        in_specs=[
            pl.BlockSpec((NC, BLK, DA), lambda i: (0, i, 0)),
            pl.BlockSpec((BLK, D), lambda i: (i, 0)),
            pl.BlockSpec((D, D), lambda i: (0, 0)),
            pl.BlockSpec((D, D), lambda i: (0, 0)),
            pl.BlockSpec((1, D), lambda i: (0, 0)),
        ],
        out_specs=pl.BlockSpec((BLK, D), lambda i: (i, 0)),
        out_shape=jax.ShapeDtypeStruct((N_NODES, D), jnp.float32),
    )(sums, x, wl_t, wr_t, bl)


def kernel(x, edge_index, W_l, b_l, W_r):
    src = edge_index[0]
    dst = edge_index[1]
    pad = E_PAD - E_EDGES
    src_p = jnp.concatenate([src, jnp.zeros((pad,), jnp.int32)])
    dst_p = jnp.concatenate([dst, jnp.full((pad,), N_NODES, jnp.int32)])
    src_r = src_p.reshape(NW, NCH, CHUNK)
    dst_r = dst_p.reshape(NW, NCH, CHUNK)

    xa = jnp.concatenate(
        [x, jnp.ones((N_NODES, 1), jnp.float32),
         jnp.zeros((N_NODES, DA - D - 1), jnp.float32)], axis=1)
    zeros = jnp.zeros((ZPT, DA), jnp.float32)

    sums = _sc_aggregate(xa, src_r, dst_r, zeros)

    return _combine(sums, x, W_l.T, W_r.T, b_l.reshape(1, D))


# speedup vs baseline: 1.2764x; 1.2764x over previous
import functools

import jax
import jax.numpy as jnp
from jax import lax
from jax.experimental import pallas as pl
from jax.experimental.pallas import tpu as pltpu
from jax.experimental.pallas import tpu_sc as plsc

N_NODES = 10000
D = 128
DA = 136
E_EDGES = 320000

NC = 2
NS = 16
NW = NC * NS
CHUNK = 64
NBUF = 2
NCH = 158
EPT = NCH * CHUNK
E_PAD = EPT * NW
ACC_N = 10112
ZPT = ACC_N // NS

_vector_mesh = plsc.VectorSubcoreMesh(core_axis_name="c", subcore_axis_name="s")


@functools.partial(
    pl.kernel,
    out_type=jax.ShapeDtypeStruct((NC, ACC_N, DA), jnp.float32),
    mesh=_vector_mesh,
    scratch_types=[
        pltpu.VMEM((NCH, CHUNK), jnp.int32),
        pltpu.VMEM((NCH, CHUNK), jnp.int32),
        pltpu.VMEM((NBUF, CHUNK, DA), jnp.float32),
        pltpu.VMEM_SHARED((ACC_N, DA), jnp.float32),
        pltpu.SemaphoreType.DMA,
        pltpu.SemaphoreType.DMA,
    ],
    compiler_params=pltpu.CompilerParams(use_tc_tiling_on_sc=False),
)
def _sc_aggregate(xa_hbm, src_hbm, dst_hbm, zero_hbm, sum_hbm,
                  src_v, dst_v, rows_r, acc_sh, sg0, sg1):
    sg = (sg0, sg1)
    cid = lax.axis_index("c")
    sid = lax.axis_index("s")
    wid = cid * NS + sid

    zbase = sid * ZPT
    pltpu.sync_copy(zero_hbm, acc_sh.at[pl.ds(zbase, ZPT)])
    pltpu.sync_copy(src_hbm.at[wid], src_v)
    pltpu.sync_copy(dst_hbm.at[wid], dst_v)

    pltpu.async_copy(xa_hbm.at[src_v.at[0]], rows_r.at[0], sg[0])

    plsc.subcore_barrier()

    @pl.loop(0, NCH, step=NBUF)
    def _(j):
        for b in range(NBUF):
            c = j + b
            nb = 1 - b
            pltpu.make_async_copy(
                xa_hbm.at[src_v.at[c]], rows_r.at[b], sg[b]).wait()

            @pl.when(c + 1 < NCH)
            def _():
                pltpu.async_copy(
                    xa_hbm.at[src_v.at[c + 1]], rows_r.at[nb], sg[nb])

            pltpu.sync_copy(rows_r.at[b], acc_sh.at[dst_v.at[c]], add=True)

    plsc.subcore_barrier()

    pltpu.sync_copy(acc_sh.at[pl.ds(zbase, ZPT)],
                    sum_hbm.at[cid].at[pl.ds(zbase, ZPT)])


BLK = 400


def _combine_body(sum_ref, x_ref, wl_ref, wr_ref, bl_ref, o_ref):
    a = sum_ref[0] + sum_ref[1]
    s = a[:, :D]
    c = a[:, D:D + 1]
    mean = s / jnp.maximum(c, 1.0)
    o_ref[...] = (
        jnp.dot(mean, wl_ref[...], preferred_element_type=jnp.float32)
        + jnp.dot(x_ref[...], wr_ref[...], preferred_element_type=jnp.float32)
        + bl_ref[...]
    )


def _combine(sums, x, wl_t, wr_t, bl):
    return pl.pallas_call(
        _combine_body,
        grid=(N_NODES // BLK,),
        in_specs=[
            pl.BlockSpec((NC, BLK, DA), lambda i: (0, i, 0)),
            pl.BlockSpec((BLK, D), lambda i: (i, 0)),
            pl.BlockSpec((D, D), lambda i: (0, 0)),
            pl.BlockSpec((D, D), lambda i: (0, 0)),
            pl.BlockSpec((1, D), lambda i: (0, 0)),
        ],
        out_specs=pl.BlockSpec((BLK, D), lambda i: (i, 0)),
        out_shape=jax.ShapeDtypeStruct((N_NODES, D), jnp.float32),
    )(sums, x, wl_t, wr_t, bl)


def kernel(x, edge_index, W_l, b_l, W_r):
    src = edge_index[0]
    dst = edge_index[1]
    pad = E_PAD - E_EDGES
    src_p = jnp.concatenate([src, jnp.zeros((pad,), jnp.int32)])
    dst_p = jnp.concatenate([dst, jnp.full((pad,), N_NODES, jnp.int32)])
    src_r = src_p.reshape(NW, NCH, CHUNK)
    dst_r = dst_p.reshape(NW, NCH, CHUNK)

    xa = jnp.concatenate(
        [x, jnp.ones((N_NODES, 1), jnp.float32),
         jnp.zeros((N_NODES, DA - D - 1), jnp.float32)], axis=1)
    zeros = jnp.zeros((ZPT, DA), jnp.float32)

    sums = _sc_aggregate(xa, src_r, dst_r, zeros)

    return _combine(sums, x, W_l.T, W_r.T, b_l.reshape(1, D))
